# Initial kernel scaffold; baseline (speedup 1.0000x reference)
import jax
import jax.numpy as jnp
from jax.experimental import pallas as pl


def kernel(inputs):
    raise NotImplementedError("write your pallas kernel here")


# speedup vs baseline: 2.6041x; 2.6041x over previous
import functools

import jax
import jax.numpy as jnp
from jax.experimental import pallas as pl
from jax.experimental.pallas import tpu as pltpu

_K_FRAC = 0.25
_MIN32 = -(2 ** 31)
_MAG = 0x7FFFFFFF


def _rows_body(k, x_ref, ts_ref, tot_ref):
    x = x_ref[...]
    bits = jax.lax.bitcast_convert_type(x, jnp.int32)
    key = bits ^ (jax.lax.shift_right_arithmetic(bits, 31) & _MAG)
    r = x.shape[0]

    def step(i, prefix):
        shift = 31 - i
        cand = prefix | jax.lax.shift_left(jnp.ones((), jnp.int32), shift)
        thresh = cand ^ _MIN32
        cnt = jnp.sum((key >= thresh).astype(jnp.int32), axis=1, keepdims=True)
        return jnp.where(cnt >= k, cand, prefix)

    prefix = jax.lax.fori_loop(0, 32, step, jnp.zeros((r, 1), jnp.int32))
    t_key = prefix ^ _MIN32
    t_bits = t_key ^ (jax.lax.shift_right_arithmetic(t_key, 31) & _MAG)
    t_val = jax.lax.bitcast_convert_type(t_bits, jnp.float32)

    gt = key > t_key
    cnt_gt = jnp.sum(gt.astype(jnp.int32), axis=1, keepdims=True)
    sum_gt = jnp.sum(jnp.where(gt, x, 0.0), axis=1, keepdims=True)
    ts_ref[...] = sum_gt + (k - cnt_gt).astype(jnp.float32) * t_val
    tot_ref[...] = jnp.sum(x, axis=1, keepdims=True)


def _finalize_body(k, hw, ts_ref, tot_ref, out_ref):
    ts = ts_ref[...]
    tot = tot_ref[...]
    ts_sum = jnp.sum(ts, axis=1, keepdims=True)
    tot_sum = jnp.sum(tot, axis=1, keepdims=True)
    out_ref[...] = ts * (tot_sum / (jnp.float32(hw) * ts_sum))


def kernel(inputs):
    b, c, h, w = inputs.shape
    hw = h * w
    k = int(_K_FRAC * hw)
    nrows = b * c
    x = inputs.reshape(nrows, hw)

    blk = 128
    grid = nrows // blk
    ts, tot = pl.pallas_call(
        functools.partial(_rows_body, k),
        grid=(grid,),
        in_specs=[pl.BlockSpec((blk, hw), lambda i: (i, 0))],
        out_specs=[
            pl.BlockSpec((blk, 1), lambda i: (i, 0)),
            pl.BlockSpec((blk, 1), lambda i: (i, 0)),
        ],
        out_shape=[
            jax.ShapeDtypeStruct((nrows, 1), jnp.float32),
            jax.ShapeDtypeStruct((nrows, 1), jnp.float32),
        ],
    )(x)

    ts = ts.reshape(b, c)
    tot = tot.reshape(b, c)
    out = pl.pallas_call(
        functools.partial(_finalize_body, k, hw),
        out_shape=jax.ShapeDtypeStruct((b, c), jnp.float32),
    )(ts, tot)
    return out


# speedup vs baseline: 5.0385x; 1.9348x over previous
import functools

import jax
import jax.numpy as jnp
from jax.experimental import pallas as pl
from jax.experimental.pallas import tpu as pltpu

_K_FRAC = 0.25
_MIN32 = -(2 ** 31)
_MAG = 0x7FFFFFFF


def _rows_body(k, xt_ref, ts_ref, tot_ref):
    x = xt_ref[...]
    bits = jax.lax.bitcast_convert_type(x, jnp.int32)
    key = bits ^ (jax.lax.shift_right_arithmetic(bits, 31) & _MAG)
    r = x.shape[1]

    def step(i, prefix):
        shift = 31 - i
        cand = prefix | jax.lax.shift_left(jnp.ones((), jnp.int32), shift)
        thresh = cand ^ _MIN32
        cnt = jnp.sum((key >= thresh).astype(jnp.int32), axis=0, keepdims=True)
        return jnp.where(cnt >= k, cand, prefix)

    prefix = jax.lax.fori_loop(0, 32, step, jnp.zeros((1, r), jnp.int32))
    t_key = prefix ^ _MIN32
    t_bits = t_key ^ (jax.lax.shift_right_arithmetic(t_key, 31) & _MAG)
    t_val = jax.lax.bitcast_convert_type(t_bits, jnp.float32)

    gt = key > t_key
    cnt_gt = jnp.sum(gt.astype(jnp.int32), axis=0, keepdims=True)
    sum_gt = jnp.sum(jnp.where(gt, x, 0.0), axis=0, keepdims=True)
    ts_ref[...] = sum_gt + (k - cnt_gt).astype(jnp.float32) * t_val
    tot_ref[...] = jnp.sum(x, axis=0, keepdims=True)


def _finalize_body(k, hw, ts_ref, tot_ref, out_ref):
    ts = ts_ref[...]
    tot = tot_ref[...]
    ts_sum = jnp.sum(ts, axis=1, keepdims=True)
    tot_sum = jnp.sum(tot, axis=1, keepdims=True)
    out_ref[...] = ts * (tot_sum / (jnp.float32(hw) * ts_sum))


def kernel(inputs):
    b, c, h, w = inputs.shape
    hw = h * w
    k = int(_K_FRAC * hw)
    nrows = b * c
    xt = inputs.reshape(nrows, hw).T

    blk = 256
    grid = nrows // blk
    ts, tot = pl.pallas_call(
        functools.partial(_rows_body, k),
        grid=(grid,),
        in_specs=[pl.BlockSpec((hw, blk), lambda i: (0, i))],
        out_specs=[
            pl.BlockSpec((1, blk), lambda i: (0, i)),
            pl.BlockSpec((1, blk), lambda i: (0, i)),
        ],
        out_shape=[
            jax.ShapeDtypeStruct((1, nrows), jnp.float32),
            jax.ShapeDtypeStruct((1, nrows), jnp.float32),
        ],
    )(xt)

    ts = ts.reshape(b, c)
    tot = tot.reshape(b, c)
    out = pl.pallas_call(
        functools.partial(_finalize_body, k, hw),
        out_shape=jax.ShapeDtypeStruct((b, c), jnp.float32),
    )(ts, tot)
    return out


# speedup vs baseline: 5.9842x; 1.1877x over previous
import functools

import jax
import jax.numpy as jnp
from jax.experimental import pallas as pl
from jax.experimental.pallas import tpu as pltpu

_K_FRAC = 0.25
_MIN32 = -(2 ** 31)
_MAG = 0x7FFFFFFF


def _rows_body(k, xt_ref, ts_ref, tot_ref):
    x = xt_ref[...]
    bits = jax.lax.bitcast_convert_type(x, jnp.int32)
    key = bits ^ (jax.lax.shift_right_arithmetic(bits, 31) & _MAG)
    r = x.shape[1]

    def step(i, prefix):
        shift = 31 - i
        cand = prefix | jax.lax.shift_left(jnp.ones((), jnp.int32), shift)
        thresh = cand ^ _MIN32
        cnt = jnp.sum((key >= thresh).astype(jnp.int32), axis=0, keepdims=True)
        return jnp.where(cnt >= k, cand, prefix)

    prefix = jax.lax.fori_loop(0, 32, step, jnp.zeros((1, r), jnp.int32))
    t_key = prefix ^ _MIN32
    t_bits = t_key ^ (jax.lax.shift_right_arithmetic(t_key, 31) & _MAG)
    t_val = jax.lax.bitcast_convert_type(t_bits, jnp.float32)

    gt = key > t_key
    cnt_gt = jnp.sum(gt.astype(jnp.int32), axis=0, keepdims=True)
    sum_gt = jnp.sum(jnp.where(gt, x, 0.0), axis=0, keepdims=True)
    ts_ref[...] = sum_gt + (k - cnt_gt).astype(jnp.float32) * t_val
    tot_ref[...] = jnp.sum(x, axis=0, keepdims=True)


def _finalize_body(k, hw, ts_ref, tot_ref, out_ref):
    ts = ts_ref[...]
    tot = tot_ref[...]
    ts_sum = jnp.sum(ts, axis=1, keepdims=True)
    tot_sum = jnp.sum(tot, axis=1, keepdims=True)
    out_ref[...] = ts * (tot_sum / (jnp.float32(hw) * ts_sum))


def kernel(inputs):
    b, c, h, w = inputs.shape
    hw = h * w
    k = int(_K_FRAC * hw)
    nrows = b * c
    xt = inputs.reshape(nrows, hw).T

    blk = 512
    grid = nrows // blk
    ts, tot = pl.pallas_call(
        functools.partial(_rows_body, k),
        grid=(grid,),
        in_specs=[pl.BlockSpec((hw, blk), lambda i: (0, i))],
        out_specs=[
            pl.BlockSpec((1, blk), lambda i: (0, i)),
            pl.BlockSpec((1, blk), lambda i: (0, i)),
        ],
        out_shape=[
            jax.ShapeDtypeStruct((1, nrows), jnp.float32),
            jax.ShapeDtypeStruct((1, nrows), jnp.float32),
        ],
    )(xt)

    ts = ts.reshape(b, c)
    tot = tot.reshape(b, c)
    out = pl.pallas_call(
        functools.partial(_finalize_body, k, hw),
        out_shape=jax.ShapeDtypeStruct((b, c), jnp.float32),
    )(ts, tot)
    return out
